# Initial kernel scaffold; baseline (speedup 1.0000x reference)
import jax
import jax.numpy as jnp
from jax.experimental import pallas as pl


def kernel(node_features, W1, b1, g1, be1, W2, b2, g2, be2, Wq, Wk):
    raise NotImplementedError("write your pallas kernel here")


# speedup vs baseline: 6.0847x; 6.0847x over previous
import functools
import math
import jax
import jax.numpy as jnp
from jax.experimental import pallas as pl
from jax.experimental.pallas import tpu as pltpu


_ERFC_SMALL = [
    +5.61802298761904239654541015625e-04,
    -4.91381669417023658752441406250e-03,
    +2.67075151205062866210937500000e-02,
    -1.12800106406211853027343750000e-01,
    +3.76122951507568359375000000000e-01,
    -1.12837910652160644531250000000e+00,
]
_ERFC_LARGE_P = [
    1.0208116471767425537109375e-01,
    4.2920666933059692382812500e-01,
    3.2379078865051269531250000e-01,
    5.3971976041793823242187500e-02,
]
_ERFC_LARGE_Q = [
    1.7251677811145782470703125e-02,
    3.9137163758277893066406250e-01,
    1.0000000000000000000000000e+00,
    6.2173241376876831054687500e-01,
    9.5662862062454223632812500e-02,
]


def _poly(x, coeffs):
    acc = jnp.full_like(x, jnp.float32(coeffs[0]))
    for c in coeffs[1:]:
        acc = acc * x + jnp.float32(c)
    return acc


def _erfc(z):
    small = z * _poly(z * z, _ERFC_SMALL) + 1.0
    a = jnp.minimum(jnp.abs(z), jnp.float32(10.06))
    q2 = 1.0 / (a * a)
    r = jnp.exp(-(a * a)) * _poly(q2, _ERFC_LARGE_P) / (a * _poly(q2, _ERFC_LARGE_Q))
    large = jnp.where(z < 0, 2.0 - r, r)
    return jnp.where(jnp.abs(z) > 1.0, large, small)


def _gelu(x):
    sqrt_half = jnp.float32(math.sqrt(0.5))
    return 0.5 * x * (1.0 - jax.lax.erf(-x * sqrt_half))


def _norm(t, mu, var, g, b, eps=1e-5):
    return (t - mu) / jnp.sqrt(var + eps) * g + b


def _dot(a, b):
    return jax.lax.dot_general(
        a, b, (((1,), (0,)), ((), ())), preferred_element_type=jnp.float32
    )


def _p1_body(x_ref, w1t, b1, o_ref):
    o_ref[...] = _dot(x_ref[...], w1t[...]) + b1[...]


def _p2_body(t1_ref, mu_ref, var_ref, g1, be1, w2t, b2, o_ref):
    xn = _norm(t1_ref[...], mu_ref[...], var_ref[...], g1[...], be1[...])
    o_ref[...] = _dot(_gelu(xn), w2t[...]) + b2[...]


def _p3_body(t2_ref, mu_ref, var_ref, g2, be2, wqt, wkt, h_ref, q_ref, k_ref):
    h = _gelu(_norm(t2_ref[...], mu_ref[...], var_ref[...], g2[...], be2[...]))
    h_ref[...] = h
    q_ref[...] = _dot(h, wqt[...])
    k_ref[...] = _dot(h, wkt[...])


def _topk_body(bm, n, topk, sqrt_g, q_ref, kt_ref, o_ref, w_ref):
    i = pl.program_id(1)
    s = _dot(q_ref[...], kt_ref[...]) / sqrt_g
    c = jax.lax.broadcasted_iota(jnp.int32, (bm, n), 1)
    r = jax.lax.broadcasted_iota(jnp.int32, (bm, n), 0) + i * bm
    w = jax.nn.sigmoid(s)
    w = jnp.where(c == r, -1.0, w)
    w_ref[...] = w
    o_ref[...] = jnp.zeros((bm, n), jnp.float32)
    for _ in range(topk):
        wv = w_ref[...]
        m = jnp.max(wv, axis=1, keepdims=True)
        cand = jnp.where(wv == m, c, n)
        fidx = jnp.min(cand, axis=1, keepdims=True)
        hit = c == fidx
        o_ref[...] = jnp.where(hit, m, o_ref[...])
        w_ref[...] = jnp.where(hit, -2.0, wv)


def _block(total, want):
    bm = min(want, total)
    while total % bm:
        bm -= 1
    return bm


def _full(shape):
    return pl.BlockSpec(shape, lambda i: (0, 0))


def kernel(node_features, W1, b1, g1, be1, W2, b2, g2, be2, Wq, Wk):
    B, N, D = node_features.shape
    H = W1.shape[0]
    G = W2.shape[0]
    BN = B * N
    topk = min(max(1, 20), max(1, N - 1))

    x2 = node_features.reshape(BN, D)
    row2 = lambda v: v.reshape(1, -1)
    bm1 = _block(BN, 256)
    g1d = (BN // bm1,)

    def rows(shape):
        return pl.BlockSpec(shape, lambda i: (i, 0))

    t1 = pl.pallas_call(
        _p1_body,
        grid=g1d,
        in_specs=[rows((bm1, D)), _full((D, H)), _full((1, H))],
        out_specs=rows((bm1, H)),
        out_shape=jax.ShapeDtypeStruct((BN, H), jnp.float32),
    )(x2, W1.T, row2(b1))

    mu1 = jnp.mean(t1, axis=-1, keepdims=True)
    var1 = jnp.mean((t1 - mu1) ** 2, axis=-1, keepdims=True)

    t2 = pl.pallas_call(
        _p2_body,
        grid=g1d,
        in_specs=[rows((bm1, H)), rows((bm1, 1)), rows((bm1, 1)),
                  _full((1, H)), _full((1, H)), _full((H, G)), _full((1, G))],
        out_specs=rows((bm1, G)),
        out_shape=jax.ShapeDtypeStruct((BN, G), jnp.float32),
    )(t1, mu1, var1, row2(g1), row2(be1), W2.T, row2(b2))

    mu2 = jnp.mean(t2, axis=-1, keepdims=True)
    var2 = jnp.mean((t2 - mu2) ** 2, axis=-1, keepdims=True)

    h2, q2, k2 = pl.pallas_call(
        _p3_body,
        grid=g1d,
        in_specs=[rows((bm1, G)), rows((bm1, 1)), rows((bm1, 1)),
                  _full((1, G)), _full((1, G)), _full((G, G)), _full((G, G))],
        out_specs=[rows((bm1, G))] * 3,
        out_shape=[jax.ShapeDtypeStruct((BN, G), jnp.float32)] * 3,
    )(t2, mu2, var2, row2(g2), row2(be2), Wq.T, Wk.T)

    kt = k2.reshape(B, N, G).transpose(0, 2, 1).reshape(B * G, N)

    bm2 = _block(N, 256)
    ni = N // bm2
    adj2 = pl.pallas_call(
        functools.partial(_topk_body, bm2, N, topk, math.sqrt(max(G, 1))),
        grid=(B, ni),
        in_specs=[
            pl.BlockSpec((bm2, G), lambda b, i, ni=ni: (b * ni + i, 0)),
            pl.BlockSpec((G, N), lambda b, i: (b, 0)),
        ],
        out_specs=pl.BlockSpec((bm2, N), lambda b, i, ni=ni: (b * ni + i, 0)),
        out_shape=jax.ShapeDtypeStruct((BN, N), jnp.float32),
        scratch_shapes=[pltpu.VMEM((bm2, N), jnp.float32)],
    )(q2, kt)

    return adj2.reshape(B, N, N), h2.reshape(B, N, G)
